# baseline (speedup 1.0000x reference)
import functools

import jax
import jax.numpy as jnp
from jax.experimental import pallas as pl
from jax.experimental.pallas import tpu as pltpu

N_OUT = 255
TILE_M = 10240


def _cde_kernel(x_ref, w_ref, b_ref, o_ref):
    acc = jnp.dot(x_ref[...], w_ref[...], preferred_element_type=jnp.float32)
    o_ref[...] = (acc + b_ref[...])[:, : o_ref.shape[-1]].astype(o_ref.dtype)


@jax.jit
def _forward(x, w_t_pad, b_pad):
    B, d_in = x.shape
    n_pad = w_t_pad.shape[1]
    tm = min(TILE_M, B)
    grid = (pl.cdiv(B, tm),)
    return pl.pallas_call(
        _cde_kernel,
        out_shape=jax.ShapeDtypeStruct((B, N_OUT), x.dtype),
        grid=grid,
        in_specs=[
            pl.BlockSpec((tm, d_in), lambda i: (i, 0)),
            pl.BlockSpec((d_in, n_pad), lambda i: (0, 0)),
            pl.BlockSpec((1, n_pad), lambda i: (0, 0)),
        ],
        out_specs=pl.BlockSpec((tm, N_OUT), lambda i: (i, 0)),
        compiler_params=pltpu.CompilerParams(
            dimension_semantics=("parallel",),
        ),
    )(x, w_t_pad, b_pad)


def kernel(x, w_t_pad, b_pad):
    return _forward(x, w_t_pad, b_pad)


# speedup vs baseline: 1.0137x; 1.0137x over previous
import functools

import jax
import jax.numpy as jnp
from jax.experimental import pallas as pl
from jax.experimental.pallas import tpu as pltpu

N_OUT = 255
TILE_M = 12288


def _cde_kernel(x_ref, w_ref, b_ref, o_ref):
    acc = jnp.dot(x_ref[...], w_ref[...], preferred_element_type=jnp.float32)
    o_ref[...] = (acc + b_ref[...])[:, : o_ref.shape[-1]].astype(o_ref.dtype)


@jax.jit
def _forward(x, w_t_pad, b_pad):
    B, d_in = x.shape
    n_pad = w_t_pad.shape[1]
    tm = min(TILE_M, B)
    grid = (pl.cdiv(B, tm),)
    return pl.pallas_call(
        _cde_kernel,
        out_shape=jax.ShapeDtypeStruct((B, N_OUT), x.dtype),
        grid=grid,
        in_specs=[
            pl.BlockSpec((tm, d_in), lambda i: (i, 0)),
            pl.BlockSpec((d_in, n_pad), lambda i: (0, 0)),
            pl.BlockSpec((1, n_pad), lambda i: (0, 0)),
        ],
        out_specs=pl.BlockSpec((tm, N_OUT), lambda i: (i, 0)),
        compiler_params=pltpu.CompilerParams(
            dimension_semantics=("parallel",),
        ),
    )(x, w_t_pad, b_pad)


def kernel(x, w_t_pad, b_pad):
    return _forward(x, w_t_pad, b_pad)
